# Initial kernel scaffold; baseline (speedup 1.0000x reference)
import jax
import jax.numpy as jnp
from jax.experimental import pallas as pl


def kernel(x, indices, values, bias):
    raise NotImplementedError("write your pallas kernel here")


# speedup vs baseline: 17.4295x; 17.4295x over previous
import functools

import jax
import jax.numpy as jnp
from jax import lax
from jax.experimental import pallas as pl
from jax.experimental.pallas import tpu as pltpu
from jax.experimental.pallas import tpu_sc as plsc

NC = 2
NS = 16
L = 16

C = 8192
SPLITS = 4
NBPG = 4


def _make_sc_kernel(B, N, M, Epad):
    EQ = Epad // SPLITS
    nch = EQ // C
    mesh = plsc.VectorSubcoreMesh(core_axis_name="c", subcore_axis_name="s")

    @functools.partial(
        pl.kernel,
        out_type=jax.ShapeDtypeStruct((B, M), jnp.float32),
        mesh=mesh,
        compiler_params=pltpu.CompilerParams(needs_layout_passes=False),
        scratch_types=[
            pltpu.VMEM((N,), jnp.float32),
            pltpu.VMEM((N,), jnp.float32),
            pltpu.VMEM((M,), jnp.float32),
            pltpu.VMEM((M,), jnp.float32),
            pltpu.VMEM((C,), jnp.int32),
            pltpu.VMEM((C,), jnp.int32),
            pltpu.VMEM((C,), jnp.float32),
            pltpu.VMEM_SHARED((NS, 2, M), jnp.float32),
        ],
    )
    def body(xf, srch, dsth, valh, biash, out, x0, x1, a0, a1, sv, dv, vv, shr):
        c = lax.axis_index("c")
        s = lax.axis_index("s")
        q = s // SPLITS
        h = s % SPLITS
        b0 = c * (2 * NBPG) + 2 * q
        is_owner = h == 0

        pltpu.sync_copy(xf.at[b0], x0)
        pltpu.sync_copy(xf.at[b0 + 1], x1)

        @pl.when(is_owner)
        def _():
            pltpu.sync_copy(biash, a0)
            pltpu.sync_copy(biash, a1)

        @pl.when(jnp.logical_not(is_owner))
        def _():
            zero = jnp.zeros((L,), jnp.float32)

            def zbody(i, carry):
                a0[pl.ds(i * L, L)] = zero
                a1[pl.ds(i * L, L)] = zero
                return carry

            lax.fori_loop(0, M // L, zbody, 0)

        def chunk_body(g, carry):
            off = h * EQ + g * C
            pltpu.sync_copy(srch.at[pl.ds(off, C)], sv)
            pltpu.sync_copy(dsth.at[pl.ds(off, C)], dv)
            pltpu.sync_copy(valh.at[pl.ds(off, C)], vv)

            def grp(j, carry2):
                isrc = sv[pl.ds(j * L, L)]
                idst = dv[pl.ds(j * L, L)]
                w = vv[pl.ds(j * L, L)]
                g0 = plsc.load_gather(x0, [isrc])
                plsc.addupdate_scatter(a0, [idst], w * g0)
                g1 = plsc.load_gather(x1, [isrc])
                plsc.addupdate_scatter(a1, [idst], w * g1)
                return carry2

            lax.fori_loop(0, C // L, grp, 0)
            return carry

        lax.fori_loop(0, nch, chunk_body, 0)

        @pl.when(jnp.logical_not(is_owner))
        def _():
            pltpu.sync_copy(a0, shr.at[s, 0])
            pltpu.sync_copy(a1, shr.at[s, 1])

        plsc.subcore_barrier()

        @pl.when(is_owner)
        def _():
            for hh in range(1, SPLITS):
                pltpu.sync_copy(shr.at[s + hh, 0], x0)
                pltpu.sync_copy(shr.at[s + hh, 1], x1)

                def addbody(i, carry):
                    o = pl.ds(i * L, L)
                    a0[o] = a0[o] + x0[o]
                    a1[o] = a1[o] + x1[o]
                    return carry

                lax.fori_loop(0, M // L, addbody, 0)
            pltpu.sync_copy(a0, out.at[b0])
            pltpu.sync_copy(a1, out.at[b0 + 1])

    return body


def kernel(x, indices, values, bias):
    B, N, _ = x.shape
    M = bias.shape[0]
    E = values.shape[0]

    xf = x[:, :, 0]
    src = indices[0]
    dst = indices[1]

    gran = SPLITS * C
    Epad = ((E + gran - 1) // gran) * gran
    pad = Epad - E
    if pad:
        src = jnp.pad(src, (0, pad))
        dst = jnp.pad(dst, (0, pad))
        values = jnp.pad(values, (0, pad))

    out = _make_sc_kernel(B, N, M, Epad)(xf, src, dst, values, bias[:, 0])
    return out[:, :, None]


# speedup vs baseline: 39.1665x; 2.2471x over previous
import functools

import jax
import jax.numpy as jnp
from jax import lax
from jax.experimental import pallas as pl
from jax.experimental.pallas import tpu as pltpu
from jax.experimental.pallas import tpu_sc as plsc

NC = 2
NS = 16
L = 16

C = 4096
SPLITS = 4
NBPG = 4
UNROLL = 8


def _make_sc_kernel(B, N, M, Epad):
    EQ = Epad // SPLITS
    nch = EQ // C
    assert nch % 2 == 0
    mesh = plsc.VectorSubcoreMesh(core_axis_name="c", subcore_axis_name="s")

    @functools.partial(
        pl.kernel,
        out_type=jax.ShapeDtypeStruct((B, M), jnp.float32),
        mesh=mesh,
        compiler_params=pltpu.CompilerParams(needs_layout_passes=False),
        scratch_types=[
            pltpu.VMEM((N,), jnp.float32),
            pltpu.VMEM((N,), jnp.float32),
            pltpu.VMEM((M,), jnp.float32),
            pltpu.VMEM((M,), jnp.float32),
            pltpu.VMEM((2, C), jnp.int32),
            pltpu.VMEM((2, C), jnp.int32),
            pltpu.VMEM((2, C), jnp.float32),
            pltpu.SemaphoreType.DMA((2,)),
            pltpu.VMEM_SHARED((NS, 2, M), jnp.float32),
        ],
    )
    def body(xf, srch, dsth, valh, biash, out, x0, x1, a0, a1, sv, dv, vv,
             sems, shr):
        c = lax.axis_index("c")
        s = lax.axis_index("s")
        q = s // SPLITS
        h = s % SPLITS
        b0 = c * (2 * NBPG) + 2 * q
        is_owner = h == 0
        e_base = h * EQ

        def start_chunk(slot, off):
            pltpu.async_copy(srch.at[pl.ds(off, C)], sv.at[slot], sems.at[slot])
            pltpu.async_copy(dsth.at[pl.ds(off, C)], dv.at[slot], sems.at[slot])
            pltpu.async_copy(valh.at[pl.ds(off, C)], vv.at[slot], sems.at[slot])

        def wait_chunk(slot):
            pltpu.make_async_copy(srch.at[pl.ds(e_base, C)], sv.at[slot],
                                  sems.at[slot]).wait()
            pltpu.make_async_copy(dsth.at[pl.ds(e_base, C)], dv.at[slot],
                                  sems.at[slot]).wait()
            pltpu.make_async_copy(valh.at[pl.ds(e_base, C)], vv.at[slot],
                                  sems.at[slot]).wait()

        start_chunk(0, e_base)

        pltpu.sync_copy(xf.at[b0], x0)
        pltpu.sync_copy(xf.at[b0 + 1], x1)

        @pl.when(is_owner)
        def _():
            pltpu.sync_copy(biash, a0)
            pltpu.sync_copy(biash, a1)

        @pl.when(jnp.logical_not(is_owner))
        def _():
            zero = jnp.zeros((L,), jnp.float32)

            @plsc.parallel_loop(0, M // L, unroll=4)
            def _(i):
                a0[pl.ds(i * L, L)] = zero
                a1[pl.ds(i * L, L)] = zero

        def chunk_body(gp, carry):
            for sl in range(2):
                g = 2 * gp + sl

                @pl.when(g + 1 < nch)
                def _():
                    start_chunk(1 - sl, e_base + (g + 1) * C)

                wait_chunk(sl)

                @plsc.parallel_loop(0, C // L, unroll=UNROLL)
                def _(j):
                    o = pl.ds(j * L, L)
                    isrc = sv[sl, o]
                    idst = dv[sl, o]
                    w = vv[sl, o]
                    g0 = plsc.load_gather(x0, [isrc])
                    plsc.addupdate_scatter(a0, [idst], w * g0)
                    g1 = plsc.load_gather(x1, [isrc])
                    plsc.addupdate_scatter(a1, [idst], w * g1)

            return carry

        lax.fori_loop(0, nch // 2, chunk_body, 0)

        @pl.when(jnp.logical_not(is_owner))
        def _():
            pltpu.sync_copy(a0, shr.at[s, 0])
            pltpu.sync_copy(a1, shr.at[s, 1])

        plsc.subcore_barrier()

        @pl.when(is_owner)
        def _():
            for hh in range(1, SPLITS):
                pltpu.sync_copy(shr.at[s + hh, 0], x0)
                pltpu.sync_copy(shr.at[s + hh, 1], x1)

                @plsc.parallel_loop(0, M // L, unroll=4)
                def _(i):
                    o = pl.ds(i * L, L)
                    a0[o] = a0[o] + x0[o]
                    a1[o] = a1[o] + x1[o]

            pltpu.sync_copy(a0, out.at[b0])
            pltpu.sync_copy(a1, out.at[b0 + 1])

    return body


def kernel(x, indices, values, bias):
    B, N, _ = x.shape
    M = bias.shape[0]
    E = values.shape[0]

    xf = x[:, :, 0]
    src = indices[0]
    dst = indices[1]

    gran = SPLITS * C * 2
    Epad = ((E + gran - 1) // gran) * gran
    pad = Epad - E
    if pad:
        src = jnp.pad(src, (0, pad))
        dst = jnp.pad(dst, (0, pad))
        values = jnp.pad(values, (0, pad))

    out = _make_sc_kernel(B, N, M, Epad)(xf, src, dst, values, bias[:, 0])
    return out[:, :, None]


# speedup vs baseline: 56.0495x; 1.4311x over previous
import functools

import jax
import jax.numpy as jnp
from jax import lax
from jax.experimental import pallas as pl
from jax.experimental.pallas import tpu as pltpu
from jax.experimental.pallas import tpu_sc as plsc

NC = 2
NS = 16
L = 16

C = 4096
SPLITS = 8
NB = 4
UNROLL = 8
RCH = 128


def _make_sc_kernel(B, N, M, Epad):
    E8 = Epad // SPLITS
    nch = E8 // C
    assert nch % 2 == 0
    MR = M // L
    nrt = NB * MR // RCH
    mesh = plsc.VectorSubcoreMesh(core_axis_name="c", subcore_axis_name="s")

    @functools.partial(
        pl.kernel,
        out_type=jax.ShapeDtypeStruct((B, MR, L), jnp.float32),
        mesh=mesh,
        compiler_params=pltpu.CompilerParams(
            needs_layout_passes=False, use_tc_tiling_on_sc=False),
        scratch_types=[
            pltpu.VMEM((N,), jnp.int32),
            pltpu.VMEM((N,), jnp.int32),
            pltpu.VMEM((MR, L), jnp.float32),
            pltpu.VMEM((MR, L), jnp.float32),
            pltpu.VMEM((MR, L), jnp.float32),
            pltpu.VMEM((MR, L), jnp.float32),
            pltpu.VMEM((2, C), jnp.int32),
            pltpu.VMEM((2, C), jnp.float32),
            pltpu.VMEM((nrt, RCH), jnp.int32),
            pltpu.SemaphoreType.DMA((2,)),
            pltpu.SemaphoreType.DMA,
            pltpu.VMEM_SHARED((NB * MR, L), jnp.float32),
            pltpu.VMEM_SHARED((NB * MR, L), jnp.float32),
        ],
    )
    def body(xph, pkh, wh, biash, idnh, out, xp0, xp1, a0, a1, a2, a3,
             pkv, wv, idv, sems, rsem, shr0, shr1):
        c = lax.axis_index("c")
        s = lax.axis_index("s")
        g = s // SPLITS
        h = s % SPLITS
        b0 = c * (2 * NB) + g * NB
        k0 = b0 // 2
        is_owner = h == 0
        e_base = h * E8
        accs = (a0, a1, a2, a3)

        def start_chunk(slot, off):
            pltpu.async_copy(pkh.at[pl.ds(off, C)], pkv.at[slot], sems.at[slot])
            pltpu.async_copy(wh.at[pl.ds(off, C)], wv.at[slot], sems.at[slot])

        def wait_chunk(slot):
            pltpu.make_async_copy(pkh.at[pl.ds(e_base, C)], pkv.at[slot],
                                  sems.at[slot]).wait()
            pltpu.make_async_copy(wh.at[pl.ds(e_base, C)], wv.at[slot],
                                  sems.at[slot]).wait()

        start_chunk(0, e_base)
        pltpu.sync_copy(xph.at[k0], xp0)
        pltpu.sync_copy(xph.at[k0 + 1], xp1)
        pltpu.sync_copy(idnh, idv)

        @pl.when(jnp.logical_and(is_owner, g == 0))
        def _():
            for bb in range(NB):
                pltpu.sync_copy(biash, shr0.at[pl.ds(bb * MR, MR)])

        @pl.when(jnp.logical_and(is_owner, g == 1))
        def _():
            for bb in range(NB):
                pltpu.sync_copy(biash, shr1.at[pl.ds(bb * MR, MR)])

        zero = jnp.zeros((L,), jnp.float32)

        @plsc.parallel_loop(0, MR, unroll=4)
        def _(i):
            a0[i, :] = zero
            a1[i, :] = zero
            a2[i, :] = zero
            a3[i, :] = zero

        plsc.subcore_barrier()

        def chunk_body(gp, carry):
            for sl in range(2):
                gg = 2 * gp + sl

                @pl.when(gg + 1 < nch)
                def _():
                    start_chunk(1 - sl, e_base + (gg + 1) * C)

                wait_chunk(sl)

                @plsc.parallel_loop(0, C // L, unroll=UNROLL)
                def _(j):
                    o = pl.ds(j * L, L)
                    p = pkv[sl, o]
                    w = wv[sl, o]
                    isrc = p & 0x3FFF
                    irow = p >> 18
                    icol = (p >> 14) & 0xF
                    xw0 = plsc.load_gather(xp0, [isrc])
                    xw1 = plsc.load_gather(xp1, [isrc])
                    f0 = plsc.bitcast(xw0 << 16, jnp.float32)
                    f1 = plsc.bitcast(xw0 & -65536, jnp.float32)
                    f2 = plsc.bitcast(xw1 << 16, jnp.float32)
                    f3 = plsc.bitcast(xw1 & -65536, jnp.float32)
                    plsc.addupdate_scatter(a0, [irow, icol], w * f0)
                    plsc.addupdate_scatter(a1, [irow, icol], w * f1)
                    plsc.addupdate_scatter(a2, [irow, icol], w * f2)
                    plsc.addupdate_scatter(a3, [irow, icol], w * f3)

            return carry

        lax.fori_loop(0, nch // 2, chunk_body, 0)

        def reduce_into(shr):
            copies = []
            for t in range(nrt):
                bb = t // (MR // RCH)
                r0 = (t % (MR // RCH)) * RCH
                copies.append(pltpu.async_copy(
                    accs[bb].at[pl.ds(r0, RCH)], shr.at[idv.at[t]], rsem,
                    add=True))
            for cp in copies:
                cp.wait()

        @pl.when(g == 0)
        def _():
            reduce_into(shr0)

        @pl.when(g == 1)
        def _():
            reduce_into(shr1)

        plsc.subcore_barrier()

        @pl.when(jnp.logical_and(is_owner, g == 0))
        def _():
            for bb in range(NB):
                pltpu.sync_copy(shr0.at[pl.ds(bb * MR, MR)], out.at[b0 + bb])

        @pl.when(jnp.logical_and(is_owner, g == 1))
        def _():
            for bb in range(NB):
                pltpu.sync_copy(shr1.at[pl.ds(bb * MR, MR)], out.at[b0 + bb])

    return body


def kernel(x, indices, values, bias):
    B, N, _ = x.shape
    M = bias.shape[0]
    E = values.shape[0]

    xb = lax.bitcast_convert_type(
        x[:, :, 0].astype(jnp.bfloat16).reshape(B // 2, 2, N), jnp.uint16
    ).astype(jnp.uint32)
    xp = lax.bitcast_convert_type(xb[:, 0] | (xb[:, 1] << 16), jnp.int32)

    gran = SPLITS * C * 2
    Epad = ((E + gran - 1) // gran) * gran
    pad = Epad - E
    pk = jnp.pad(indices[0] | (indices[1] << 14), (0, pad))
    w = jnp.pad(values, (0, pad))

    MR = M // L
    idn = jnp.arange(4 * MR, dtype=jnp.int32).reshape(-1, RCH)

    out = _make_sc_kernel(B, N, M, Epad)(
        xp, pk, w, bias.reshape(MR, L), idn)
    return out.reshape(B, M)[:, :, None]
